# baseline (speedup 1.0000x reference)
import jax
import jax.numpy as jnp
from jax import lax
from jax.experimental import pallas as pl
from jax.experimental.pallas import tpu as pltpu
from jax.experimental.pallas import tpu_sc as plsc

N_NODES = 100000
N_AUG = N_NODES + 1
B = 4
E = 3200000
NC = 2
NS = 16
NW = NC * NS
BINS = 32
BIN_R = 4096
N_X = BINS * BIN_R
EW = E // NW
KP = 2000
NCHP = EW // KP
KR = 4096
E2P = 2 * E + 2 * BINS * KR
N_STEPS = 10
DT = 1.0 / N_STEPS

_i32 = jnp.int32
_f32 = jnp.float32

_mesh = plsc.VectorSubcoreMesh(core_axis_name="c", subcore_axis_name="s",
                               num_cores=NC, num_subcores=NS)
_params = pltpu.CompilerParams(needs_layout_passes=False,
                               use_tc_tiling_on_sc=False)


def _iota16():
  return lax.iota(_i32, 16)


def _wid():
  return lax.axis_index("c") * NS + lax.axis_index("s")


def _sread(ref, idx):
  return plsc.load_gather(ref, [lax.full((16,), idx, _i32)])[0]


def _hist_body(src_hbm, des_hbm, hist_out, sib, dib, hcnt):
  w = _wid()
  z16 = jnp.zeros((16,), _i32)
  hcnt[pl.ds(0, 16)] = z16
  hcnt[pl.ds(16, 16)] = z16

  def chunk(k, _):
    eoff = w * EW + k * KP
    pltpu.sync_copy(src_hbm.at[pl.ds(eoff, KP)], sib)
    pltpu.sync_copy(des_hbm.at[pl.ds(eoff, KP)], dib)

    def vloop(i, _):
      sl = pl.ds(i * 16, 16)
      for ref in (sib, dib):
        binv = lax.shift_right_logical(ref[sl], 12)
        cnt, last = plsc.scan_count(binv)
        plsc.addupdate_scatter(hcnt, [binv], cnt, mask=last)
      return 0

    lax.fori_loop(0, KP // 16, vloop, 0)
    return 0

  lax.fori_loop(0, NCHP, chunk, 0)
  pltpu.sync_copy(hcnt, hist_out.at[w])


_hist_call = pl.kernel(
    _hist_body,
    out_type=jax.ShapeDtypeStruct((NW, BINS), _i32),
    mesh=_mesh,
    compiler_params=_params,
    scratch_types=[
        pltpu.VMEM((KP,), _i32),
        pltpu.VMEM((KP,), _i32),
        pltpu.VMEM((BINS,), _i32),
    ],
)


def _part_body(src_hbm, des_hbm, g_hbm, base_hbm, spad_hbm, cend_hbm,
               recs_out, sgs_out,
               sib, dib, gib, posb, recb, counters, zrec, zsg, p16,
               spadm, cendm):
  w = _wid()
  pltpu.sync_copy(base_hbm.at[w], counters)
  pltpu.sync_copy(spad_hbm, spadm)
  pltpu.sync_copy(cend_hbm, cendm)

  def chunk(k, _):
    eoff = w * EW + k * KP
    pltpu.sync_copy(src_hbm.at[pl.ds(eoff, KP)], sib)
    pltpu.sync_copy(des_hbm.at[pl.ds(eoff, KP)], dib)
    pltpu.sync_copy(g_hbm.at[pl.ds(eoff, KP)], gib)
    for dstref, othref in ((sib, dib), (dib, sib)):
      def vloop(i, _):
        sl = pl.ds(i * 16, 16)
        dstv = dstref[sl]
        othv = othref[sl]
        binv = lax.shift_right_logical(dstv, 12)
        rec = lax.shift_left(othv, 12) | (dstv & 4095)
        old = plsc.load_gather(counters, [binv])
        cnt, last = plsc.scan_count(binv)
        posb[sl] = old + cnt - 1
        recb[sl] = rec
        plsc.addupdate_scatter(counters, [binv], cnt, mask=last)
        return 0

      lax.fori_loop(0, KP // 16, vloop, 0)
      pltpu.sync_copy(recb, recs_out.at[posb])
      pltpu.sync_copy(gib, sgs_out.at[posb])
    return 0

  lax.fori_loop(0, NCHP, chunk, 0)

  def zloop(j, _):
    sl = pl.ds(j * 16, 16)
    zrec[sl] = jnp.zeros((16,), _i32)
    zsg[sl] = jnp.zeros((16,), _f32)
    return 0

  lax.fori_loop(0, KR // 16, zloop, 0)

  c0 = _sread(cendm, w)
  p1 = _sread(spadm, w + 1)
  p16[pl.ds(0, 16)] = c0 + _iota16()
  pltpu.sync_copy(zrec.at[pl.ds(0, 16)], recs_out.at[p16])
  pltpu.sync_copy(zsg.at[pl.ds(0, 16)], sgs_out.at[p16])
  pe = pl.multiple_of(p1 - KR, 8)
  pltpu.sync_copy(zrec, recs_out.at[pl.ds(pe, KR)])
  pltpu.sync_copy(zsg, sgs_out.at[pl.ds(pe, KR)])
  c8 = pl.multiple_of((c0 + 7) & ~7, 8)

  @pl.when(c8 + KR <= p1)
  def _():
    pltpu.sync_copy(zrec, recs_out.at[pl.ds(c8, KR)])
    pltpu.sync_copy(zsg, sgs_out.at[pl.ds(c8, KR)])


_part_call = pl.kernel(
    _part_body,
    out_type=(jax.ShapeDtypeStruct((E2P,), _i32),
              jax.ShapeDtypeStruct((E2P,), _f32)),
    mesh=_mesh,
    compiler_params=_params,
    scratch_types=[
        pltpu.VMEM((KP,), _i32),
        pltpu.VMEM((KP,), _i32),
        pltpu.VMEM((KP,), _f32),
        pltpu.VMEM((KP,), _i32),
        pltpu.VMEM((KP,), _i32),
        pltpu.VMEM((BINS,), _i32),
        pltpu.VMEM((KR,), _i32),
        pltpu.VMEM((KR,), _f32),
        pltpu.VMEM((16,), _i32),
        pltpu.VMEM((BINS + 8,), _i32),
        pltpu.VMEM((BINS,), _i32),
    ],
)


def _deg_body(recs_hbm, sgs_hbm, spad_hbm, deg_out, recb, sgb, dacc, spadm):
  b = _wid()
  pltpu.sync_copy(spad_hbm, spadm)

  def zloop(j, _):
    dacc[pl.ds(j * 16, 16)] = jnp.zeros((16,), _f32)
    return 0

  lax.fori_loop(0, BIN_R // 16, zloop, 0)

  r0 = _sread(spadm, b)
  nch = (_sread(spadm, b + 1) - r0) // KR

  def chunk(k, _):
    off = pl.multiple_of(r0 + k * KR, 8)
    pltpu.sync_copy(recs_hbm.at[pl.ds(off, KR)], recb)
    pltpu.sync_copy(sgs_hbm.at[pl.ds(off, KR)], sgb)

    def vloop(i, _):
      sl = pl.ds(i * 16, 16)
      dstv = recb[sl] & 4095
      plsc.addupdate_scatter(dacc, [dstv], sgb[sl])
      return 0

    lax.fori_loop(0, KR // 16, vloop, 0)
    return 0

  lax.fori_loop(0, nch, chunk, 0)
  pltpu.sync_copy(dacc, deg_out.at[pl.ds(pl.multiple_of(b * BIN_R, 8), BIN_R)])


_deg_call = pl.kernel(
    _deg_body,
    out_type=jax.ShapeDtypeStruct((N_X,), _f32),
    mesh=_mesh,
    compiler_params=_params,
    scratch_types=[
        pltpu.VMEM((KR,), _i32),
        pltpu.VMEM((KR,), _f32),
        pltpu.VMEM((BIN_R,), _f32),
        pltpu.VMEM((BINS + 8,), _i32),
    ],
)


def _step_body(x_hbm, recs_hbm, sgs_hbm, spad_hbm, deg_hbm, xout_hbm,
               recb, sgb, oid4b, vrows, acc, xb, degb, spadm):
  b = _wid()
  pltpu.sync_copy(spad_hbm, spadm)
  q4 = lax.shift_right_logical(_iota16(), 2)
  m4 = _iota16() & 3
  zf = jnp.zeros((16,), _f32)

  def azloop(j, _):
    acc[pl.ds(j * 16, 16)] = zf
    return 0

  lax.fori_loop(0, BIN_R * B // 16, azloop, 0, unroll=4)

  r0 = _sread(spadm, b)
  nch = (_sread(spadm, b + 1) - r0) // KR

  def chunk(k, _):
    off = pl.multiple_of(r0 + k * KR, 8)
    pltpu.sync_copy(recs_hbm.at[pl.ds(off, KR)], recb)
    pltpu.sync_copy(sgs_hbm.at[pl.ds(off, KR)], sgb)

    def oloop(i, _):
      sl = pl.ds(i * 16, 16)
      rec4 = plsc.load_gather(recb, [q4 + i * 4])
      oid = lax.shift_right_logical(rec4, 12)
      oid4b[sl] = lax.shift_left(oid, 2) | m4
      return 0

    lax.fori_loop(0, KR * B // 16, oloop, 0, unroll=4)
    pltpu.sync_copy(x_hbm.at[oid4b], vrows)

    def vloop(i, _):
      sl = pl.ds(i * 16, 16)
      rec4 = plsc.load_gather(recb, [q4 + i * 4])
      dst4 = rec4 & 4095
      sg4 = plsc.load_gather(sgb, [q4 + i * 4])
      cur = sg4 * vrows[sl]
      plsc.addupdate_scatter(acc, [lax.shift_left(dst4, 2) | m4], cur)
      return 0

    lax.fori_loop(0, KR * B // 16, vloop, 0, unroll=4)
    return 0

  lax.fori_loop(0, nch, chunk, 0)

  row0 = pl.multiple_of(b * BIN_R * B, 8)
  pltpu.sync_copy(x_hbm.at[pl.ds(row0, BIN_R * B)], xb)
  pltpu.sync_copy(deg_hbm.at[pl.ds(pl.multiple_of(b * BIN_R, 8), BIN_R)], degb)

  def uloop(j, _):
    sl = pl.ds(j * 16, 16)
    v = xb[sl]
    dg = plsc.load_gather(degb, [q4 + j * 4])
    xb[sl] = v + DT * (acc[sl] - v * dg)
    return 0

  lax.fori_loop(0, BIN_R * B // 16, uloop, 0, unroll=4)

  @pl.when(b == 0)
  def _():
    v0 = xb[pl.ds(0, 16)]
    xb[pl.ds(0, 16)] = jnp.where(_iota16() >= B, v0, 0.0)

  pltpu.sync_copy(xb, xout_hbm.at[pl.ds(row0, BIN_R * B)])


_step_call = pl.kernel(
    _step_body,
    out_type=jax.ShapeDtypeStruct((N_X * B,), _f32),
    mesh=_mesh,
    compiler_params=_params,
    scratch_types=[
        pltpu.VMEM((KR,), _i32),
        pltpu.VMEM((KR,), _f32),
        pltpu.VMEM((KR * B,), _i32),
        pltpu.VMEM((KR * B,), _f32),
        pltpu.VMEM((BIN_R * B,), _f32),
        pltpu.VMEM((BIN_R * B,), _f32),
        pltpu.VMEM((BIN_R,), _f32),
        pltpu.VMEM((BINS + 8,), _i32),
    ],
)


@jax.jit
def kernel(x0, edge_index, edge_param):
  src = edge_index[0]
  des = edge_index[1]
  x = jnp.pad(x0.T, ((1, N_X - N_AUG), (0, 0))).reshape(-1)

  hist = _hist_call(src, des)
  total = hist.sum(axis=0)
  region = (total + (KR - 1)) // KR * KR + KR
  spad = jnp.concatenate([jnp.zeros((1,), _i32),
                          jnp.cumsum(region).astype(_i32),
                          jnp.zeros((7,), _i32)])
  excl = jnp.cumsum(hist, axis=0) - hist
  base = spad[None, :BINS] + excl
  cend = spad[:BINS] + total

  recs, sgs = _part_call(src, des, edge_param, base.astype(_i32),
                         spad.astype(_i32), cend.astype(_i32))
  deg = _deg_call(recs, sgs, spad.astype(_i32))

  for _ in range(N_STEPS):
    x = _step_call(x, recs, sgs, spad.astype(_i32), deg)
  return x.reshape(N_X, B)[1:N_AUG, :].T


# speedup vs baseline: 1.3589x; 1.3589x over previous
import jax
import jax.numpy as jnp
from jax import lax
from jax.experimental import pallas as pl
from jax.experimental.pallas import tpu as pltpu
from jax.experimental.pallas import tpu_sc as plsc

N_NODES = 100000
N_AUG = N_NODES + 1
B = 4
E = 3200000
NC = 2
NS = 16
NW = NC * NS
BINS = 32
BIN_R = 4096
N_X = BINS * BIN_R
EW = E // NW
KP = 2000
NCHP = EW // KP
KR = 1024
E2P = 2 * E + 2 * BINS * KR
N_STEPS = 10
DT = 1.0 / N_STEPS

_i32 = jnp.int32
_f32 = jnp.float32

_mesh = plsc.VectorSubcoreMesh(core_axis_name="c", subcore_axis_name="s",
                               num_cores=NC, num_subcores=NS)
_params = pltpu.CompilerParams(needs_layout_passes=False,
                               use_tc_tiling_on_sc=False)


def _iota16():
  return lax.iota(_i32, 16)


def _wid():
  return lax.axis_index("c") * NS + lax.axis_index("s")


def _sread(ref, idx):
  return plsc.load_gather(ref, [lax.full((16,), idx, _i32)])[0]


def _hist_body(src_hbm, des_hbm, hist_out, sib, dib, hcnt):
  w = _wid()
  z16 = jnp.zeros((16,), _i32)
  hcnt[pl.ds(0, 16)] = z16
  hcnt[pl.ds(16, 16)] = z16

  def chunk(k, _):
    eoff = w * EW + k * KP
    pltpu.sync_copy(src_hbm.at[pl.ds(eoff, KP)], sib)
    pltpu.sync_copy(des_hbm.at[pl.ds(eoff, KP)], dib)

    def vloop(i, _):
      sl = pl.ds(i * 16, 16)
      for ref in (sib, dib):
        binv = lax.shift_right_logical(ref[sl], 12)
        cnt, last = plsc.scan_count(binv)
        plsc.addupdate_scatter(hcnt, [binv], cnt, mask=last)
      return 0

    lax.fori_loop(0, KP // 16, vloop, 0)
    return 0

  lax.fori_loop(0, NCHP, chunk, 0)
  pltpu.sync_copy(hcnt, hist_out.at[w])


_hist_call = pl.kernel(
    _hist_body,
    out_type=jax.ShapeDtypeStruct((NW, BINS), _i32),
    mesh=_mesh,
    compiler_params=_params,
    scratch_types=[
        pltpu.VMEM((KP,), _i32),
        pltpu.VMEM((KP,), _i32),
        pltpu.VMEM((BINS,), _i32),
    ],
)


def _part_body(src_hbm, des_hbm, g_hbm, base_hbm, spad_hbm, cend_hbm,
               recs_out, sgs_out,
               sib, dib, gib, posb, recb, counters, zrec, zsg, p16,
               spadm, cendm):
  w = _wid()
  pltpu.sync_copy(base_hbm.at[w], counters)
  pltpu.sync_copy(spad_hbm, spadm)
  pltpu.sync_copy(cend_hbm, cendm)

  def chunk(k, _):
    eoff = w * EW + k * KP
    pltpu.sync_copy(src_hbm.at[pl.ds(eoff, KP)], sib)
    pltpu.sync_copy(des_hbm.at[pl.ds(eoff, KP)], dib)
    pltpu.sync_copy(g_hbm.at[pl.ds(eoff, KP)], gib)
    for dstref, othref in ((sib, dib), (dib, sib)):
      def vloop(i, _):
        sl = pl.ds(i * 16, 16)
        dstv = dstref[sl]
        othv = othref[sl]
        binv = lax.shift_right_logical(dstv, 12)
        rec = lax.shift_left(othv, 12) | (dstv & 4095)
        old = plsc.load_gather(counters, [binv])
        cnt, last = plsc.scan_count(binv)
        posb[sl] = old + cnt - 1
        recb[sl] = rec
        plsc.addupdate_scatter(counters, [binv], cnt, mask=last)
        return 0

      lax.fori_loop(0, KP // 16, vloop, 0)
      pltpu.sync_copy(recb, recs_out.at[posb])
      pltpu.sync_copy(gib, sgs_out.at[posb])
    return 0

  lax.fori_loop(0, NCHP, chunk, 0)

  def zloop(j, _):
    sl = pl.ds(j * 16, 16)
    zrec[sl] = jnp.zeros((16,), _i32)
    zsg[sl] = jnp.zeros((16,), _f32)
    return 0

  lax.fori_loop(0, KR // 16, zloop, 0)

  c0 = _sread(cendm, w)
  p1 = _sread(spadm, w + 1)
  p16[pl.ds(0, 16)] = c0 + _iota16()
  pltpu.sync_copy(zrec.at[pl.ds(0, 16)], recs_out.at[p16])
  pltpu.sync_copy(zsg.at[pl.ds(0, 16)], sgs_out.at[p16])
  pe = pl.multiple_of(p1 - KR, 8)
  pltpu.sync_copy(zrec, recs_out.at[pl.ds(pe, KR)])
  pltpu.sync_copy(zsg, sgs_out.at[pl.ds(pe, KR)])
  c8 = pl.multiple_of((c0 + 7) & ~7, 8)

  @pl.when(c8 + KR <= p1)
  def _():
    pltpu.sync_copy(zrec, recs_out.at[pl.ds(c8, KR)])
    pltpu.sync_copy(zsg, sgs_out.at[pl.ds(c8, KR)])


_part_call = pl.kernel(
    _part_body,
    out_type=(jax.ShapeDtypeStruct((E2P,), _i32),
              jax.ShapeDtypeStruct((E2P,), _f32)),
    mesh=_mesh,
    compiler_params=_params,
    scratch_types=[
        pltpu.VMEM((KP,), _i32),
        pltpu.VMEM((KP,), _i32),
        pltpu.VMEM((KP,), _f32),
        pltpu.VMEM((KP,), _i32),
        pltpu.VMEM((KP,), _i32),
        pltpu.VMEM((BINS,), _i32),
        pltpu.VMEM((KR,), _i32),
        pltpu.VMEM((KR,), _f32),
        pltpu.VMEM((16,), _i32),
        pltpu.VMEM((BINS + 8,), _i32),
        pltpu.VMEM((BINS,), _i32),
    ],
)


def _deg_body(recs_hbm, sgs_hbm, spad_hbm, deg_out, recb, sgb, dacc, spadm):
  b = _wid()
  pltpu.sync_copy(spad_hbm, spadm)

  def zloop(j, _):
    dacc[pl.ds(j * 16, 16)] = jnp.zeros((16,), _f32)
    return 0

  lax.fori_loop(0, BIN_R // 16, zloop, 0)

  r0 = _sread(spadm, b)
  nch = (_sread(spadm, b + 1) - r0) // KR

  def chunk(k, _):
    off = pl.multiple_of(r0 + k * KR, 8)
    pltpu.sync_copy(recs_hbm.at[pl.ds(off, KR)], recb)
    pltpu.sync_copy(sgs_hbm.at[pl.ds(off, KR)], sgb)

    def vloop(i, _):
      sl = pl.ds(i * 16, 16)
      dstv = recb[sl] & 4095
      plsc.addupdate_scatter(dacc, [dstv], sgb[sl])
      return 0

    lax.fori_loop(0, KR // 16, vloop, 0)
    return 0

  lax.fori_loop(0, nch, chunk, 0)
  pltpu.sync_copy(dacc, deg_out.at[pl.ds(pl.multiple_of(b * BIN_R, 8), BIN_R)])


_deg_call = pl.kernel(
    _deg_body,
    out_type=jax.ShapeDtypeStruct((N_X,), _f32),
    mesh=_mesh,
    compiler_params=_params,
    scratch_types=[
        pltpu.VMEM((KR,), _i32),
        pltpu.VMEM((KR,), _f32),
        pltpu.VMEM((BIN_R,), _f32),
        pltpu.VMEM((BINS + 8,), _i32),
    ],
)


def _step_body(x_hbm, recs_hbm, sgs_hbm, spad_hbm, deg_hbm, xout_hbm,
               recb, sgb, oid4b, vrows, acc, xb, degb, spadm):
  b = _wid()
  pltpu.sync_copy(spad_hbm, spadm)
  q4 = lax.shift_right_logical(_iota16(), 2)
  m4 = _iota16() & 3
  zf = jnp.zeros((16,), _f32)

  def azloop(j, _):
    acc[pl.ds(j * 16, 16)] = zf
    return 0

  lax.fori_loop(0, BIN_R * B // 16, azloop, 0, unroll=4)

  r0 = _sread(spadm, b)
  nch = (_sread(spadm, b + 1) - r0) // KR

  def chunk(k, _):
    off = pl.multiple_of(r0 + k * KR, 8)
    pltpu.sync_copy(recs_hbm.at[pl.ds(off, KR)], recb)
    pltpu.sync_copy(sgs_hbm.at[pl.ds(off, KR)], sgb)

    def oloop(i, _):
      sl = pl.ds(i * 16, 16)
      rec4 = plsc.load_gather(recb, [q4 + i * 4])
      oid = lax.shift_right_logical(rec4, 12)
      oid4b[sl] = lax.shift_left(oid, 2) | m4
      return 0

    lax.fori_loop(0, KR * B // 16, oloop, 0, unroll=4)
    pltpu.sync_copy(x_hbm.at[oid4b], vrows)

    def vloop(i, _):
      sl = pl.ds(i * 16, 16)
      rec4 = plsc.load_gather(recb, [q4 + i * 4])
      dst4 = rec4 & 4095
      sg4 = plsc.load_gather(sgb, [q4 + i * 4])
      cur = sg4 * vrows[sl]
      plsc.addupdate_scatter(acc, [lax.shift_left(dst4, 2) | m4], cur)
      return 0

    lax.fori_loop(0, KR * B // 16, vloop, 0, unroll=4)
    return 0

  lax.fori_loop(0, nch, chunk, 0)

  row0 = pl.multiple_of(b * BIN_R * B, 8)
  pltpu.sync_copy(x_hbm.at[pl.ds(row0, BIN_R * B)], xb)
  pltpu.sync_copy(deg_hbm.at[pl.ds(pl.multiple_of(b * BIN_R, 8), BIN_R)], degb)

  def uloop(j, _):
    sl = pl.ds(j * 16, 16)
    v = xb[sl]
    dg = plsc.load_gather(degb, [q4 + j * 4])
    xb[sl] = v + DT * (acc[sl] - v * dg)
    return 0

  lax.fori_loop(0, BIN_R * B // 16, uloop, 0, unroll=4)

  @pl.when(b == 0)
  def _():
    v0 = xb[pl.ds(0, 16)]
    xb[pl.ds(0, 16)] = jnp.where(_iota16() >= B, v0, 0.0)

  pltpu.sync_copy(xb, xout_hbm.at[pl.ds(row0, BIN_R * B)])


_step_call = pl.kernel(
    _step_body,
    out_type=jax.ShapeDtypeStruct((N_X * B,), _f32),
    mesh=_mesh,
    compiler_params=_params,
    scratch_types=[
        pltpu.VMEM((KR,), _i32),
        pltpu.VMEM((KR,), _f32),
        pltpu.VMEM((KR * B,), _i32),
        pltpu.VMEM((KR * B,), _f32),
        pltpu.VMEM((BIN_R * B,), _f32),
        pltpu.VMEM((BIN_R * B,), _f32),
        pltpu.VMEM((BIN_R,), _f32),
        pltpu.VMEM((BINS + 8,), _i32),
    ],
)


@jax.jit
def kernel(x0, edge_index, edge_param):
  src = edge_index[0]
  des = edge_index[1]
  x = jnp.pad(x0.T, ((1, N_X - N_AUG), (0, 0))).reshape(-1)

  hist = _hist_call(src, des)
  total = hist.sum(axis=0)
  region = (total + (KR - 1)) // KR * KR + KR
  spad = jnp.concatenate([jnp.zeros((1,), _i32),
                          jnp.cumsum(region).astype(_i32),
                          jnp.zeros((7,), _i32)])
  excl = jnp.cumsum(hist, axis=0) - hist
  base = spad[None, :BINS] + excl
  cend = spad[:BINS] + total

  recs, sgs = _part_call(src, des, edge_param, base.astype(_i32),
                         spad.astype(_i32), cend.astype(_i32))
  deg = _deg_call(recs, sgs, spad.astype(_i32))

  for _ in range(N_STEPS):
    x = _step_call(x, recs, sgs, spad.astype(_i32), deg)
  return x.reshape(N_X, B)[1:N_AUG, :].T
